# baseline (speedup 1.0000x reference)
import functools

import jax
import jax.numpy as jnp
from jax import lax
from jax.experimental import pallas as pl
from jax.experimental.pallas import tpu as pltpu


def _fused_kernel(batch_ref, x_ref, glob_ref, w_ref, b_ref, out_ref,
                  acc_ref, *, tile_n, n_nodes):
    n = pl.program_id(0)
    n_graphs = acc_ref.shape[0]
    h = x_ref.shape[1]

    @pl.when(n == 0)
    def _init():
        acc_ref[...] = jnp.zeros_like(acc_ref)

    x_t = x_ref[...]
    if n_nodes % tile_n != 0:
        row_ids = n * tile_n + lax.broadcasted_iota(jnp.int32, (tile_n, 1), 0)
        x_t = jnp.where(row_ids < n_nodes, x_t, 0)

    ids = batch_ref[...]
    graph_iota = lax.broadcasted_iota(jnp.int32, (n_graphs, tile_n), 0)
    mask = (ids == graph_iota).astype(jnp.bfloat16)

    acc_ref[...] += jnp.dot(mask, x_t.astype(jnp.bfloat16),
                            preferred_element_type=jnp.float32)

    @pl.when(n == pl.num_programs(0) - 1)
    def _finalize():
        w = w_ref[...]
        dn = (((1,), (1,)), ((), ()))
        out = (lax.dot_general(glob_ref[...], w[:, :h], dn,
                               preferred_element_type=jnp.float32)
               + lax.dot_general(acc_ref[...], w[:, h:], dn,
                                 preferred_element_type=jnp.float32)
               + b_ref[...])
        out_ref[...] = out.astype(out_ref.dtype)


def kernel(x, glob, batch, weight, bias):
    n_nodes, h = x.shape
    b_graphs = glob.shape[0]
    out_dtype = jnp.result_type(x.dtype, glob.dtype, weight.dtype)

    tile_n = min(4096, max(128, ((n_nodes + 127) // 128) * 128))
    steps = pl.cdiv(n_nodes, tile_n)

    batch2d = batch.astype(jnp.int32).reshape(1, n_nodes)
    bias2d = bias.reshape(1, h)

    out = pl.pallas_call(
        functools.partial(_fused_kernel, tile_n=tile_n, n_nodes=n_nodes),
        out_shape=jax.ShapeDtypeStruct((b_graphs, h), out_dtype),
        grid=(steps,),
        in_specs=[
            pl.BlockSpec((1, tile_n), lambda n: (0, n)),
            pl.BlockSpec((tile_n, h), lambda n: (n, 0)),
            pl.BlockSpec((b_graphs, h), lambda n: (0, 0)),
            pl.BlockSpec((h, 2 * h), lambda n: (0, 0)),
            pl.BlockSpec((1, h), lambda n: (0, 0)),
        ],
        out_specs=pl.BlockSpec((b_graphs, h), lambda n: (0, 0)),
        scratch_shapes=[pltpu.VMEM((b_graphs, h), jnp.float32)],
        compiler_params=pltpu.CompilerParams(
            dimension_semantics=("arbitrary",),
        ),
        cost_estimate=pl.CostEstimate(
            flops=2 * b_graphs * n_nodes * h + 4 * b_graphs * h * h,
            transcendentals=0,
            bytes_accessed=n_nodes * h * x.dtype.itemsize + n_nodes * 4
                           + 2 * h * h * weight.dtype.itemsize
                           + 2 * b_graphs * h * 4,
        ),
    )(batch2d, x, glob, weight, bias2d)

    return out


# speedup vs baseline: 1.1547x; 1.1547x over previous
import functools

import jax
import jax.numpy as jnp
from jax import lax
from jax.experimental import pallas as pl
from jax.experimental.pallas import tpu as pltpu


def _fused_kernel(x_ref, batch_ref, glob_ref, w_ref, b_ref, out_ref,
                  acc_ref, *, tile_n, n_nodes):
    n = pl.program_id(0)
    n_graphs = acc_ref.shape[0]
    h = x_ref.shape[1]

    @pl.when(n == 0)
    def _init():
        acc_ref[...] = jnp.zeros_like(acc_ref)

    x_t = x_ref[...]
    if n_nodes % tile_n != 0:
        row_ids = n * tile_n + lax.broadcasted_iota(jnp.int32, (tile_n, 1), 0)
        x_t = jnp.where(row_ids < n_nodes, x_t, 0)

    ids = batch_ref[...]
    graph_iota = lax.broadcasted_iota(jnp.int32, (n_graphs, tile_n), 0)
    mask = (ids == graph_iota).astype(jnp.bfloat16)

    acc_ref[...] += jnp.dot(mask, x_t.astype(jnp.bfloat16),
                            preferred_element_type=jnp.float32)

    @pl.when(n == pl.num_programs(0) - 1)
    def _finalize():
        w = w_ref[...]
        dn = (((1,), (1,)), ((), ()))
        out = (lax.dot_general(glob_ref[...], w[:, :h], dn,
                               preferred_element_type=jnp.float32)
               + lax.dot_general(acc_ref[...], w[:, h:], dn,
                                 preferred_element_type=jnp.float32)
               + b_ref[...])
        out_ref[...] = out.astype(out_ref.dtype)


def kernel(x, glob, batch, weight, bias):
    n_nodes, h = x.shape
    b_graphs = glob.shape[0]
    out_dtype = jnp.result_type(x.dtype, glob.dtype, weight.dtype)

    tile_n = min(8192, max(128, ((n_nodes + 127) // 128) * 128))
    steps = pl.cdiv(n_nodes, tile_n)

    batch2d = batch.astype(jnp.int32).reshape(1, n_nodes)
    bias2d = bias.reshape(1, h)

    out = pl.pallas_call(
        functools.partial(_fused_kernel, tile_n=tile_n, n_nodes=n_nodes),
        out_shape=jax.ShapeDtypeStruct((b_graphs, h), out_dtype),
        grid=(steps,),
        in_specs=[
            pl.BlockSpec((tile_n, h), lambda n: (n, 0)),
            pl.BlockSpec((1, tile_n), lambda n: (0, n)),
            pl.BlockSpec((b_graphs, h), lambda n: (0, 0)),
            pl.BlockSpec((h, 2 * h), lambda n: (0, 0)),
            pl.BlockSpec((1, h), lambda n: (0, 0)),
        ],
        out_specs=pl.BlockSpec((b_graphs, h), lambda n: (0, 0)),
        scratch_shapes=[pltpu.VMEM((b_graphs, h), jnp.float32)],
        compiler_params=pltpu.CompilerParams(
            dimension_semantics=("arbitrary",),
        ),
        cost_estimate=pl.CostEstimate(
            flops=2 * b_graphs * n_nodes * h + 4 * b_graphs * h * h,
            transcendentals=0,
            bytes_accessed=n_nodes * h * x.dtype.itemsize + n_nodes * 4
                           + 2 * h * h * weight.dtype.itemsize
                           + 2 * b_graphs * h * 4,
        ),
    )(x, batch2d, glob, weight, bias2d)

    return out
